# Initial kernel scaffold; baseline (speedup 1.0000x reference)
import jax
import jax.numpy as jnp
from jax.experimental import pallas as pl


def kernel(task_feature, answers, W1, b1, Wmu, bmu, Wls, bls, W2, b2, W3, b3, worker_mu, worker_rho):
    raise NotImplementedError("write your pallas kernel here")


# speedup vs baseline: 2.7285x; 2.7285x over previous
import functools

import jax
import jax.numpy as jnp
from jax import lax
from jax.experimental import pallas as pl
from jax.experimental.pallas import tpu as pltpu
from jax.experimental.pallas import tpu_sc as plsc

TASK_NUM = 100000
FEATURE = 128
WORKER = 10000
CLASS = 10
CP = 16
HIDDEN = 256
ANSWERS = 500000

NC, NS = 2, 16
NW = NC * NS
APAD = 512000
B_PER_W = APAD // NW
CH = 128
NCH = B_PER_W // CH

BN = 2000
GRID = TASK_NUM // BN


def _dense_body(tf_ref, W1_ref, b1_ref, Wmuls_ref, bmuls_ref, eps_ref,
                W2_ref, b2_ref, W3_ref, b3_ref, rho_ref, muw_ref,
                z_ref, mu_ref, ls_ref, rec_ref, spmu_ref):
    isq = 1.0 / jnp.sqrt(jnp.float32(1.0) + jnp.float32(1e-3))
    tf = tf_ref[...]
    h = jnp.maximum(
        jnp.dot(tf, W1_ref[...], preferred_element_type=jnp.float32) * isq
        + b1_ref[...], 0.0)
    muls = jnp.dot(h, Wmuls_ref[...], preferred_element_type=jnp.float32) \
        + bmuls_ref[...]
    mu = muls[:, :CP]
    ls = muls[:, CP:]
    z = mu + eps_ref[...] * jnp.exp(ls)
    z_ref[...] = z
    mu_ref[...] = mu
    ls_ref[...] = ls
    x = jnp.maximum(
        jnp.dot(z, W2_ref[...], preferred_element_type=jnp.float32)
        + b2_ref[...], 0.0) * isq
    rec_ref[...] = jnp.dot(x, W3_ref[...], preferred_element_type=jnp.float32) \
        + b3_ref[...]
    @pl.when(pl.program_id(0) == 0)
    def _():
        rho = rho_ref[...]
        sp = jnp.maximum(rho, 0.0) + jnp.log1p(jnp.exp(-jnp.abs(rho)))
        spmu_ref[...] = jnp.concatenate([sp, muw_ref[...]], axis=1)


def _dense_call(tf, W1, b1, Wmuls, bmuls, eps_p, W2p, b2, W3, b3, rho_p, muw_p):
    f32 = jnp.float32
    const2 = lambda shape: pl.BlockSpec(shape, lambda i: (0, 0))
    return pl.pallas_call(
        _dense_body,
        grid=(GRID,),
        in_specs=[
            pl.BlockSpec((BN, FEATURE), lambda i: (i, 0)),
            const2((FEATURE, HIDDEN)),
            const2((1, HIDDEN)),
            const2((HIDDEN, 2 * CP)),
            const2((1, 2 * CP)),
            pl.BlockSpec((BN, CP), lambda i: (i, 0)),
            const2((CP, HIDDEN)),
            const2((1, HIDDEN)),
            const2((HIDDEN, FEATURE)),
            const2((1, FEATURE)),
            const2((WORKER, CP)),
            const2((WORKER, CP)),
        ],
        out_specs=[
            pl.BlockSpec((BN, CP), lambda i: (i, 0)),
            pl.BlockSpec((BN, CP), lambda i: (i, 0)),
            pl.BlockSpec((BN, CP), lambda i: (i, 0)),
            pl.BlockSpec((BN, FEATURE), lambda i: (i, 0)),
            const2((WORKER, 2 * CP)),
        ],
        out_shape=[
            jax.ShapeDtypeStruct((TASK_NUM, CP), f32),
            jax.ShapeDtypeStruct((TASK_NUM, CP), f32),
            jax.ShapeDtypeStruct((TASK_NUM, CP), f32),
            jax.ShapeDtypeStruct((TASK_NUM, FEATURE), f32),
            jax.ShapeDtypeStruct((WORKER, 2 * CP), f32),
        ],
    )(tf, W1, b1, Wmuls, bmuls, eps_p, W2p, b2, W3, b3, rho_p, muw_p)


def _sc_body(z_hbm, spmu_hbm, ridx_hbm, cidx_hbm, out_hbm,
             ridx_v, cidx_v, zrows, smrows, outv, sem):
    wid = lax.axis_index("s") * NC + lax.axis_index("c")
    pltpu.sync_copy(ridx_hbm.at[wid], ridx_v)
    pltpu.sync_copy(cidx_hbm.at[wid], cidx_v)

    def chunk(j, carry):
        cz = pltpu.async_copy(z_hbm.at[ridx_v.at[j]], zrows, sem)
        cs = pltpu.async_copy(spmu_hbm.at[cidx_v.at[j]], smrows, sem)
        cz.wait()
        cs.wait()

        def row(i, c):
            outv[i, :] = zrows[i, :] * smrows[i, 0:CP] + smrows[i, CP:2 * CP]
            return c

        lax.fori_loop(0, CH, row, 0, unroll=8)
        pltpu.sync_copy(outv, out_hbm.at[wid, pl.ds(j * CH, CH)])
        return carry

    lax.fori_loop(0, NCH, chunk, 0)


@functools.lru_cache(maxsize=1)
def _make_sc_gather():
    return pl.kernel(
        _sc_body,
        out_type=jax.ShapeDtypeStruct((NW, B_PER_W, CP), jnp.float32),
        mesh=plsc.VectorSubcoreMesh(core_axis_name="c", subcore_axis_name="s",
                                    num_cores=NC, num_subcores=NS),
        scratch_types=[
            pltpu.VMEM((NCH, CH), jnp.int32),
            pltpu.VMEM((NCH, CH), jnp.int32),
            pltpu.VMEM((CH, CP), jnp.float32),
            pltpu.VMEM((CH, 2 * CP), jnp.float32),
            pltpu.VMEM((CH, CP), jnp.float32),
            pltpu.SemaphoreType.DMA,
        ],
        compiler_params=pltpu.CompilerParams(use_tc_tiling_on_sc=False),
    )


def kernel(task_feature, answers, W1, b1, Wmu, bmu, Wls, bls, W2, b2, W3, b3,
           worker_mu, worker_rho):
    f32 = jnp.float32
    pad_c = lambda a: jnp.pad(a, ((0, 0), (0, CP - CLASS)))
    Wmuls = jnp.concatenate([pad_c(Wmu), pad_c(Wls)], axis=1)
    bmuls = jnp.concatenate(
        [jnp.pad(bmu, (0, CP - CLASS)), jnp.pad(bls, (0, CP - CLASS))]
    ).reshape(1, 2 * CP)
    eps = 0.01 * jax.random.normal(jax.random.key(1), (TASK_NUM, CLASS),
                                   dtype=f32)
    eps_p = pad_c(eps)
    W2p = jnp.pad(W2, ((0, CP - CLASS), (0, 0)))
    rho_p = pad_c(worker_rho)
    muw_p = pad_c(worker_mu)

    z_p, mu_p, ls_p, recons, spmu = _dense_call(
        task_feature, W1, b1.reshape(1, HIDDEN), Wmuls, bmuls, eps_p,
        W2p, b2.reshape(1, HIDDEN), W3, b3.reshape(1, FEATURE), rho_p, muw_p)

    ridx = jnp.pad(answers[:, 0], (0, APAD - ANSWERS)).reshape(NW, NCH, CH)
    cidx = jnp.pad(answers[:, 1], (0, APAD - ANSWERS)).reshape(NW, NCH, CH)

    crowd_p = _make_sc_gather()(z_p, spmu, ridx, cidx)
    crowd = crowd_p.reshape(APAD, CP)[:ANSWERS, :CLASS]

    return (crowd, z_p[:, :CLASS], recons, mu_p[:, :CLASS], ls_p[:, :CLASS])


# speedup vs baseline: 3.0332x; 1.1117x over previous
import functools

import jax
import jax.numpy as jnp
from jax import lax
from jax.experimental import pallas as pl
from jax.experimental.pallas import tpu as pltpu
from jax.experimental.pallas import tpu_sc as plsc

TASK_NUM = 100000
FEATURE = 128
WORKER = 10000
CLASS = 10
CP = 16
HIDDEN = 256
ANSWERS = 500000

NC, NS = 2, 16
NW = NC * NS
APAD = 512000
B_PER_W = APAD // NW
CH = 128
NCH = B_PER_W // CH

BN = 2000
GRID = TASK_NUM // BN


def _dense_body(tf_ref, W1_ref, b1_ref, Wmuls_ref, bmuls_ref, eps_ref,
                W2_ref, b2_ref, W3_ref, b3_ref, rho_ref, muw_ref,
                z_ref, mu_ref, ls_ref, rec_ref, spmu_ref):
    isq = 1.0 / jnp.sqrt(jnp.float32(1.0) + jnp.float32(1e-3))
    tf = tf_ref[...]
    h = jnp.maximum(
        jnp.dot(tf, W1_ref[...], preferred_element_type=jnp.float32) * isq
        + b1_ref[...], 0.0)
    muls = jnp.dot(h, Wmuls_ref[...], preferred_element_type=jnp.float32) \
        + bmuls_ref[...]
    mu = muls[:, :CP]
    ls = muls[:, CP:]
    z = mu + eps_ref[...] * jnp.exp(ls)
    z_ref[...] = z
    mu_ref[...] = mu
    ls_ref[...] = ls
    x = jnp.maximum(
        jnp.dot(z, W2_ref[...], preferred_element_type=jnp.float32)
        + b2_ref[...], 0.0) * isq
    rec_ref[...] = jnp.dot(x, W3_ref[...], preferred_element_type=jnp.float32) \
        + b3_ref[...]
    @pl.when(pl.program_id(0) == 0)
    def _():
        rho = rho_ref[...]
        sp = jnp.maximum(rho, 0.0) + jnp.log1p(jnp.exp(-jnp.abs(rho)))
        spmu_ref[...] = jnp.concatenate([sp, muw_ref[...]], axis=1)


def _dense_call(tf, W1, b1, Wmuls, bmuls, eps_p, W2p, b2, W3, b3, rho_p, muw_p):
    f32 = jnp.float32
    const2 = lambda shape: pl.BlockSpec(shape, lambda i: (0, 0))
    return pl.pallas_call(
        _dense_body,
        grid=(GRID,),
        in_specs=[
            pl.BlockSpec((BN, FEATURE), lambda i: (i, 0)),
            const2((FEATURE, HIDDEN)),
            const2((1, HIDDEN)),
            const2((HIDDEN, 2 * CP)),
            const2((1, 2 * CP)),
            pl.BlockSpec((BN, CP), lambda i: (i, 0)),
            const2((CP, HIDDEN)),
            const2((1, HIDDEN)),
            const2((HIDDEN, FEATURE)),
            const2((1, FEATURE)),
            const2((WORKER, CP)),
            const2((WORKER, CP)),
        ],
        out_specs=[
            pl.BlockSpec((BN, CP), lambda i: (i, 0)),
            pl.BlockSpec((BN, CP), lambda i: (i, 0)),
            pl.BlockSpec((BN, CP), lambda i: (i, 0)),
            pl.BlockSpec((BN, FEATURE), lambda i: (i, 0)),
            const2((WORKER, 2 * CP)),
        ],
        out_shape=[
            jax.ShapeDtypeStruct((TASK_NUM, CP), f32),
            jax.ShapeDtypeStruct((TASK_NUM, CP), f32),
            jax.ShapeDtypeStruct((TASK_NUM, CP), f32),
            jax.ShapeDtypeStruct((TASK_NUM, FEATURE), f32),
            jax.ShapeDtypeStruct((WORKER, 2 * CP), f32),
        ],
    )(tf, W1, b1, Wmuls, bmuls, eps_p, W2p, b2, W3, b3, rho_p, muw_p)


def _sc_body(z_hbm, spmu_hbm, ridx_hbm, cidx_hbm, out_hbm,
             ridx_v, cidx_v, zr0, zr1, sm0, sm1, outv, sem):
    wid = lax.axis_index("s") * NC + lax.axis_index("c")
    pltpu.sync_copy(ridx_hbm.at[wid], ridx_v)
    pltpu.sync_copy(cidx_hbm.at[wid], cidx_v)

    def fire(j, zb, sb):
        pltpu.async_copy(z_hbm.at[ridx_v.at[j]], zb, sem)
        pltpu.async_copy(spmu_hbm.at[cidx_v.at[j]], sb, sem)

    def process(j, zb, sb):
        pltpu.make_async_copy(z_hbm.at[ridx_v.at[j]], zb, sem).wait()
        pltpu.make_async_copy(spmu_hbm.at[cidx_v.at[j]], sb, sem).wait()

        def row(i, c):
            outv[i, :] = zb[i, :] * sb[i, 0:CP] + sb[i, CP:2 * CP]
            return c

        lax.fori_loop(0, CH, row, 0, unroll=8)
        pltpu.sync_copy(outv, out_hbm.at[wid, pl.ds(j * CH, CH)])

    fire(0, zr0, sm0)

    def pair(p, carry):
        j = 2 * p

        @pl.when(j + 1 < NCH)
        def _():
            fire(j + 1, zr1, sm1)

        process(j, zr0, sm0)

        @pl.when(j + 2 < NCH)
        def _():
            fire(j + 2, zr0, sm0)

        @pl.when(j + 1 < NCH)
        def _():
            process(j + 1, zr1, sm1)

        return carry

    lax.fori_loop(0, (NCH + 1) // 2, pair, 0)


@functools.lru_cache(maxsize=1)
def _make_sc_gather():
    return pl.kernel(
        _sc_body,
        out_type=jax.ShapeDtypeStruct((NW, B_PER_W, CP), jnp.float32),
        mesh=plsc.VectorSubcoreMesh(core_axis_name="c", subcore_axis_name="s",
                                    num_cores=NC, num_subcores=NS),
        scratch_types=[
            pltpu.VMEM((NCH, CH), jnp.int32),
            pltpu.VMEM((NCH, CH), jnp.int32),
            pltpu.VMEM((CH, CP), jnp.float32),
            pltpu.VMEM((CH, CP), jnp.float32),
            pltpu.VMEM((CH, 2 * CP), jnp.float32),
            pltpu.VMEM((CH, 2 * CP), jnp.float32),
            pltpu.VMEM((CH, CP), jnp.float32),
            pltpu.SemaphoreType.DMA,
        ],
        compiler_params=pltpu.CompilerParams(use_tc_tiling_on_sc=False),
    )


def kernel(task_feature, answers, W1, b1, Wmu, bmu, Wls, bls, W2, b2, W3, b3,
           worker_mu, worker_rho):
    f32 = jnp.float32
    pad_c = lambda a: jnp.pad(a, ((0, 0), (0, CP - CLASS)))
    Wmuls = jnp.concatenate([pad_c(Wmu), pad_c(Wls)], axis=1)
    bmuls = jnp.concatenate(
        [jnp.pad(bmu, (0, CP - CLASS)), jnp.pad(bls, (0, CP - CLASS))]
    ).reshape(1, 2 * CP)
    eps = 0.01 * jax.random.normal(jax.random.key(1), (TASK_NUM, CLASS),
                                   dtype=f32)
    eps_p = pad_c(eps)
    W2p = jnp.pad(W2, ((0, CP - CLASS), (0, 0)))
    rho_p = pad_c(worker_rho)
    muw_p = pad_c(worker_mu)

    z_p, mu_p, ls_p, recons, spmu = _dense_call(
        task_feature, W1, b1.reshape(1, HIDDEN), Wmuls, bmuls, eps_p,
        W2p, b2.reshape(1, HIDDEN), W3, b3.reshape(1, FEATURE), rho_p, muw_p)

    ridx = jnp.pad(answers[:, 0], (0, APAD - ANSWERS)).reshape(NW, NCH, CH)
    cidx = jnp.pad(answers[:, 1], (0, APAD - ANSWERS)).reshape(NW, NCH, CH)

    crowd_p = _make_sc_gather()(z_p, spmu, ridx, cidx)
    crowd = crowd_p.reshape(APAD, CP)[:ANSWERS, :CLASS]

    return (crowd, z_p[:, :CLASS], recons, mu_p[:, :CLASS], ls_p[:, :CLASS])
